# baseline (speedup 1.0000x reference)
import functools

import jax
import jax.numpy as jnp
from jax import lax
from jax.experimental import pallas as pl
from jax.experimental.pallas import tpu as pltpu
from jax.experimental.pallas import tpu_sc as plsc

_NUM_EXPERTS = 64
_TILE = 4096
_N_WORKERS = 32
_LANES = 16


def _gate_body(x_ref, w_ref, b_ref, idx_ref, topw_ref):
    t = x_ref.shape[0]
    e = _NUM_EXPERTS
    logits = jax.lax.dot_general(
        x_ref[...], w_ref[...],
        dimension_numbers=(((1,), (1,)), ((), ())),
        preferred_element_type=jnp.float32,
    ) + b_ref[...]

    iota = jax.lax.broadcasted_iota(jnp.int32, (t, e), 1)
    m0 = jnp.max(logits, axis=1, keepdims=True)
    i0 = jnp.min(jnp.where(logits == m0, iota, e), axis=1, keepdims=True)
    sel0 = iota == i0
    masked = jnp.where(sel0, -jnp.inf, logits)
    m1 = jnp.max(masked, axis=1, keepdims=True)
    i1 = jnp.min(jnp.where(masked == m1, iota, e), axis=1, keepdims=True)

    z = jnp.exp(m1 - m0)
    w0 = 1.0 / (1.0 + z)
    w1 = z / (1.0 + z)

    idx_ref[...] = jnp.concatenate([i0, i1], axis=1)
    topw_ref[...] = jnp.concatenate([w0, w1], axis=1)


def _tc_gate(x, W, b):
    n, d = x.shape
    e = _NUM_EXPERTS
    b2 = b.reshape(1, e)
    return pl.pallas_call(
        _gate_body,
        grid=(n // _TILE,),
        in_specs=[
            pl.BlockSpec((_TILE, d), lambda i: (i, 0)),
            pl.BlockSpec((e, d), lambda i: (0, 0)),
            pl.BlockSpec((1, e), lambda i: (0, 0)),
        ],
        out_specs=[
            pl.BlockSpec((_TILE, 2), lambda i: (i, 0)),
            pl.BlockSpec((_TILE, 2), lambda i: (i, 0)),
        ],
        out_shape=[
            jax.ShapeDtypeStruct((n, 2), jnp.int32),
            jax.ShapeDtypeStruct((n, 2), jnp.float32),
        ],
    )(x, W, b2)


def _make_sc_scatter(n_tokens):
    e = _NUM_EXPERTS
    tok_per_w = n_tokens // _N_WORKERS
    n_groups = (tok_per_w * 2) // _LANES
    mesh = plsc.VectorSubcoreMesh(core_axis_name="c", subcore_axis_name="s")

    @functools.partial(
        pl.kernel,
        mesh=mesh,
        out_type=jax.ShapeDtypeStruct((n_tokens, e), jnp.float32),
        compiler_params=pltpu.CompilerParams(
            needs_layout_passes=False, use_tc_tiling_on_sc=False),
        scratch_types=[
            pltpu.VMEM((tok_per_w, e), jnp.float32),
            pltpu.VMEM((tok_per_w, 2), jnp.int32),
            pltpu.VMEM((tok_per_w, 2), jnp.float32),
            pltpu.SemaphoreType.DMA,
            pltpu.SemaphoreType.DMA,
        ],
    )
    def sc_scatter(idx_hbm, w_hbm, out_hbm, out_v, idx_v, w_v, sem1, sem2):
        wid = lax.axis_index("s") * 2 + lax.axis_index("c")
        tok_base = wid * tok_per_w

        cp_idx = pltpu.async_copy(
            idx_hbm.at[pl.ds(tok_base, tok_per_w)], idx_v, sem1)
        cp_w = pltpu.async_copy(
            w_hbm.at[pl.ds(tok_base, tok_per_w)], w_v, sem2)

        zeros16 = jnp.zeros((_LANES,), jnp.float32)

        def zero_body(t, carry):
            for c in range(e // _LANES):
                out_v[t, pl.ds(c * _LANES, _LANES)] = zeros16
            return carry

        lax.fori_loop(0, tok_per_w, zero_body, 0, unroll=4)

        cp_idx.wait()
        cp_w.wait()

        half = lax.iota(jnp.int32, _LANES) >> 1

        def scatter_body(g, carry):
            t0 = g * 8
            expert = plsc.load_gather(
                idx_v, [t0 + half,
                        lax.iota(jnp.int32, _LANES) & 1])
            wvals = plsc.load_gather(
                w_v, [t0 + half,
                      lax.iota(jnp.int32, _LANES) & 1])
            plsc.store_scatter(out_v, [t0 + half, expert], wvals)
            return carry

        lax.fori_loop(0, n_groups, scatter_body, 0, unroll=4)

        pltpu.sync_copy(out_v, out_hbm.at[pl.ds(tok_base, tok_per_w)])

    return sc_scatter


def kernel(x, W, b):
    n, _ = x.shape
    idx, topw = _tc_gate(x, W, b)
    sparse = _make_sc_scatter(n)(idx, topw)
    return (sparse, idx, topw)


# speedup vs baseline: 1.8845x; 1.8845x over previous
import jax
import jax.numpy as jnp
from jax.experimental import pallas as pl
from jax.experimental.pallas import tpu as pltpu

_NUM_EXPERTS = 64
_TILE = 4096


def _gate_body(x_ref, w_ref, b_ref, sparse_ref, idx_ref, topw_ref):
    t = x_ref.shape[0]
    e = _NUM_EXPERTS
    logits = jax.lax.dot_general(
        x_ref[...], w_ref[...],
        dimension_numbers=(((1,), (1,)), ((), ())),
        preferred_element_type=jnp.float32,
    ) + b_ref[...]

    iota = jax.lax.broadcasted_iota(jnp.int32, (t, e), 1).astype(jnp.float32)
    fill = jnp.float32(e)
    m0 = jnp.max(logits, axis=1, keepdims=True)
    i0 = jnp.min(jnp.where(logits == m0, iota, fill), axis=1, keepdims=True)
    sel0 = iota == i0
    masked = jnp.where(sel0, -jnp.inf, logits)
    m1 = jnp.max(masked, axis=1, keepdims=True)
    i1 = jnp.min(jnp.where(masked == m1, iota, fill), axis=1, keepdims=True)
    sel1 = iota == i1

    z = jnp.exp(m1 - m0)
    w0 = 1.0 / (1.0 + z)
    w1 = z / (1.0 + z)

    sparse_ref[...] = jnp.where(sel0, w0, 0.0) + jnp.where(sel1, w1, 0.0)
    idx_ref[...] = jnp.concatenate([i0, i1], axis=1).astype(jnp.int32)
    topw_ref[...] = jnp.concatenate([w0, w1], axis=1)


def kernel(x, W, b):
    n, d = x.shape
    e = _NUM_EXPERTS
    grid = n // _TILE
    b2 = b.reshape(1, e)
    sparse, idx, topw = pl.pallas_call(
        _gate_body,
        grid=(grid,),
        in_specs=[
            pl.BlockSpec((_TILE, d), lambda i: (i, 0)),
            pl.BlockSpec((e, d), lambda i: (0, 0)),
            pl.BlockSpec((1, e), lambda i: (0, 0)),
        ],
        out_specs=[
            pl.BlockSpec((_TILE, e), lambda i: (i, 0)),
            pl.BlockSpec((_TILE, 2), lambda i: (i, 0)),
            pl.BlockSpec((_TILE, 2), lambda i: (i, 0)),
        ],
        out_shape=[
            jax.ShapeDtypeStruct((n, e), x.dtype),
            jax.ShapeDtypeStruct((n, 2), jnp.int32),
            jax.ShapeDtypeStruct((n, 2), jnp.float32),
        ],
    )(x, W, b2)
    return (sparse, idx, topw)
